# baseline (speedup 1.0000x reference)
import functools

import jax
import jax.numpy as jnp
from jax import lax
from jax.experimental import pallas as pl
from jax.experimental.pallas import tpu as pltpu
from jax.experimental.pallas import tpu_sc as plsc

_B = 16384
_HIST = 50
_HP = 56
_D = 64
_DP = 128
_NC = 2
_NS = 16
_NW = _NC * _NS
_BAGS_PER_W = _B // _NW
_SLAB = 128
_NSLAB = _BAGS_PER_W // _SLAB
_NREG = _D // 16
_TCOLS = 256


def _tc_bagmajor_idx(xt):
    def body(xt_ref, out_ref):
        t = xt_ref[...].T
        out_ref[...] = jnp.concatenate(
            [t, jnp.zeros((_TCOLS, _HP - _HIST), jnp.int32)], axis=1)

    return pl.pallas_call(
        body,
        grid=(_B // _TCOLS,),
        in_specs=[pl.BlockSpec((_HIST, _TCOLS), lambda i: (0, i))],
        out_specs=pl.BlockSpec((_TCOLS, _HP), lambda i: (i, 0)),
        out_shape=jax.ShapeDtypeStruct((_B, _HP), jnp.int32),
    )(xt)


def _sc_embedding_sum(xb, tp, bias16):
    mesh = plsc.VectorSubcoreMesh(
        core_axis_name="c", subcore_axis_name="s",
        num_cores=_NC, num_subcores=_NS,
    )

    @functools.partial(
        pl.kernel,
        out_type=jax.ShapeDtypeStruct((_B * _D,), jnp.float32),
        mesh=mesh,
        compiler_params=pltpu.CompilerParams(use_tc_tiling_on_sc=True),
        scratch_types=[
            pltpu.VMEM((_SLAB, _HP), jnp.int32),
            pltpu.VMEM((_HP, _DP), jnp.float32),
            pltpu.VMEM((_HP, _DP), jnp.float32),
            pltpu.VMEM((_BAGS_PER_W * _D,), jnp.float32),
            pltpu.VMEM((16, _DP), jnp.float32),
            pltpu.SemaphoreType.DMA,
            pltpu.SemaphoreType.DMA,
        ],
    )
    def k(xb_hbm, tab_hbm, bias_hbm, out_hbm,
          idx_v, rows0, rows1, out_v, bias_v, sem0, sem1):
        wid = lax.axis_index("s") * _NC + lax.axis_index("c")
        base = wid * _BAGS_PER_W
        pltpu.sync_copy(bias_hbm, bias_v)
        bias_regs = [bias_v[0, pl.ds(16 * g, 16)] for g in range(_NREG)]

        def start(b, rows, sem):
            pltpu.async_copy(tab_hbm.at[idx_v.at[b]], rows, sem)

        def wait(b, rows, sem):
            pltpu.make_async_copy(tab_hbm.at[idx_v.at[b]], rows, sem).wait()

        def reduce_bag(ob, rows):
            accs = list(bias_regs)
            for l in range(_HIST):
                accs = [accs[g] + rows[l, pl.ds(16 * g, 16)]
                        for g in range(_NREG)]
            for g in range(_NREG):
                out_v[pl.ds(_D * ob + 16 * g, 16)] = accs[g]

        for sl in range(_NSLAB):
            sbase = base + sl * _SLAB
            pltpu.sync_copy(xb_hbm.at[pl.ds(sbase, _SLAB), :], idx_v)
            start(0, rows0, sem0)

            def step(i, carry):
                b = 2 * i
                start(b + 1, rows1, sem1)
                wait(b, rows0, sem0)
                reduce_bag(sl * _SLAB + b, rows0)

                @pl.when(b + 2 < _SLAB)
                def _prefetch():
                    start(b + 2, rows0, sem0)

                wait(b + 1, rows1, sem1)
                reduce_bag(sl * _SLAB + b + 1, rows1)
                return carry

            lax.fori_loop(0, _SLAB // 2, step, 0)

        pltpu.sync_copy(out_v, out_hbm.at[pl.ds(base * _D, _BAGS_PER_W * _D)])

    return k(xb, tp, bias16)


def kernel(x, table, emb_bias):
    xt = x.astype(jnp.int32).T
    xb = _tc_bagmajor_idx(xt)
    tp = jnp.pad(table, ((0, 0), (0, _DP - _D)))
    bias_pad = jnp.pad(emb_bias, (0, _DP - _D)).reshape(1, _DP)
    bias16 = jnp.broadcast_to(bias_pad, (16, _DP))
    out = _sc_embedding_sum(xb, tp, bias16)
    return out.reshape(_B, _D)


# speedup vs baseline: 5.4932x; 5.4932x over previous
import functools

import jax
import jax.numpy as jnp
from jax import lax
from jax.experimental import pallas as pl
from jax.experimental.pallas import tpu as pltpu
from jax.experimental.pallas import tpu_sc as plsc

_B = 16384
_HIST = 50
_D = 64
_NC = 2
_NS = 16
_NW = _NC * _NS
_BAGS_PER_W = _B // _NW
_CPB = 2
_IPC = _CPB * _HIST
_CHUNKS = _BAGS_PER_W // _CPB
_NREG = _D // 16


def _sc_embedding_sum(x2d, table, emb_bias):
    mesh = plsc.VectorSubcoreMesh(
        core_axis_name="c", subcore_axis_name="s",
        num_cores=_NC, num_subcores=_NS,
    )

    @functools.partial(
        pl.kernel,
        out_type=jax.ShapeDtypeStruct((_B, _D), jnp.float32),
        mesh=mesh,
        compiler_params=pltpu.CompilerParams(use_tc_tiling_on_sc=False),
        scratch_types=[
            pltpu.VMEM((_CHUNKS, _IPC), jnp.int32),
            pltpu.VMEM((_IPC, _D), jnp.float32),
            pltpu.VMEM((_IPC, _D), jnp.float32),
            pltpu.VMEM((_IPC, _D), jnp.float32),
            pltpu.VMEM((_IPC, _D), jnp.float32),
            pltpu.VMEM((_BAGS_PER_W, _D), jnp.float32),
            pltpu.VMEM((_D,), jnp.float32),
            pltpu.SemaphoreType.DMA,
            pltpu.SemaphoreType.DMA,
            pltpu.SemaphoreType.DMA,
            pltpu.SemaphoreType.DMA,
        ],
    )
    def k(x_hbm, tab_hbm, bias_hbm, out_hbm,
          idx_v, rows0, rows1, rows2, rows3, out_v, bias_v,
          sem0, sem1, sem2, sem3):
        wid = lax.axis_index("s") * _NC + lax.axis_index("c")
        pltpu.sync_copy(x_hbm.at[pl.ds(wid * _CHUNKS, _CHUNKS)], idx_v)
        pltpu.sync_copy(bias_hbm, bias_v)
        bias_regs = [bias_v[pl.ds(16 * g, 16)] for g in range(_NREG)]

        def start(j, rows, sem):
            pltpu.async_copy(tab_hbm.at[idx_v.at[j]], rows, sem)

        def wait(j, rows, sem):
            pltpu.make_async_copy(tab_hbm.at[idx_v.at[j]], rows, sem).wait()

        def reduce_chunk(j, rows):
            for bag in range(_CPB):
                accs = list(bias_regs)
                for l in range(_HIST):
                    r = bag * _HIST + l
                    accs = [accs[g] + rows[r, pl.ds(16 * g, 16)]
                            for g in range(_NREG)]
                ob = j * _CPB + bag
                for g in range(_NREG):
                    out_v[ob, pl.ds(16 * g, 16)] = accs[g]

        bufs = (rows0, rows1, rows2, rows3)
        sems = (sem0, sem1, sem2, sem3)
        _DEPTH = 4
        for kk in range(_DEPTH - 1):
            start(kk, bufs[kk], sems[kk])

        def step(i, carry):
            base = _DEPTH * i
            for kk in range(_DEPTH):
                j = base + kk
                nxt = j + (_DEPTH - 1)

                @pl.when(nxt < _CHUNKS)
                def _prefetch():
                    start(nxt, bufs[(kk + _DEPTH - 1) % _DEPTH],
                          sems[(kk + _DEPTH - 1) % _DEPTH])

                wait(j, bufs[kk], sems[kk])
                reduce_chunk(j, bufs[kk])
            return carry

        lax.fori_loop(0, _CHUNKS // _DEPTH, step, 0)
        pltpu.sync_copy(out_v, out_hbm.at[pl.ds(wid * _BAGS_PER_W, _BAGS_PER_W)])

    return k(x2d, table, emb_bias)


def kernel(x, table, emb_bias):
    x2d = x.astype(jnp.int32).reshape(_B * _HIST // _IPC, _IPC)
    return _sc_embedding_sum(x2d, table, emb_bias)
